# baseline (device time: 853170 ns/iter reference)
import jax
import jax.numpy as jnp
from jax import lax
from jax.experimental import pallas as pl
from jax.experimental.pallas import tpu as pltpu

N_DEV = 4

_GELU_C = 0.7978845608028654


def _gelu(y):
    return 0.5 * y * (1.0 + jnp.tanh(_GELU_C * (y + 0.044715 * y * y * y)))


def kernel(x, w_mat):
    m_per, k = x.shape
    _, n_per = w_mat.shape
    half = m_per // 2
    blk_n = 512
    n_tiles = n_per // blk_n

    def body(x_ref, w_ref, out_ref, xf_ref,
             copy_sem, cw_send, cw_recv, ccw_send, ccw_recv):
        my_pos = lax.axis_index("i")
        left = (my_pos - 1) % N_DEV
        right = (my_pos + 1) % N_DEV

        barrier_sem = pltpu.get_barrier_semaphore()
        for nbr in [left, right]:
            pl.semaphore_signal(
                barrier_sem, inc=1,
                device_id=(nbr,), device_id_type=pl.DeviceIdType.MESH,
            )
        pl.semaphore_wait(barrier_sem, 2)

        cp = pltpu.make_async_copy(
            x_ref, xf_ref.at[pl.ds(my_pos * m_per, m_per), :], copy_sem
        )
        cp.start()
        cp.wait()

        def inner(x_blk, w_blk, o_blk):
            y = jnp.dot(
                x_blk[...], w_blk[...], preferred_element_type=jnp.float32
            )
            o_blk[...] = _gelu(y)

        def compute_half(row_blk):
            pipe = pltpu.emit_pipeline(
                inner,
                grid=(1, n_tiles),
                in_specs=[
                    pl.BlockSpec((half, k), lambda i, j: (row_blk, 0)),
                    pl.BlockSpec((k, blk_n), lambda i, j: (0, j)),
                ],
                out_specs=[pl.BlockSpec((half, blk_n), lambda i, j: (row_blk, j))],
            )
            pipe(xf_ref, w_ref, out_ref)

        def start_hop(h):
            o_cw = (my_pos - h) % N_DEV
            o_ccw = (my_pos + h) % N_DEV
            rdma_cw = pltpu.make_async_remote_copy(
                src_ref=xf_ref.at[pl.ds(o_cw * m_per, half), :],
                dst_ref=xf_ref.at[pl.ds(o_cw * m_per, half), :],
                send_sem=cw_send.at[h],
                recv_sem=cw_recv.at[h],
                device_id=(right,),
                device_id_type=pl.DeviceIdType.MESH,
            )
            rdma_ccw = pltpu.make_async_remote_copy(
                src_ref=xf_ref.at[pl.ds(o_ccw * m_per + half, half), :],
                dst_ref=xf_ref.at[pl.ds(o_ccw * m_per + half, half), :],
                send_sem=ccw_send.at[h],
                recv_sem=ccw_recv.at[h],
                device_id=(left,),
                device_id_type=pl.DeviceIdType.MESH,
            )
            rdma_cw.start()
            rdma_ccw.start()
            return rdma_cw, rdma_ccw

        for h in range(N_DEV - 1):
            rdma_cw, rdma_ccw = start_hop(h)
            if h == 0:
                compute_half(my_pos * 2)
                compute_half(my_pos * 2 + 1)
            else:
                compute_half(((my_pos - h) % N_DEV) * 2)
                compute_half(((my_pos + h) % N_DEV) * 2 + 1)
            rdma_cw.wait()
            rdma_ccw.wait()

        compute_half(((my_pos - (N_DEV - 1)) % N_DEV) * 2)
        compute_half(((my_pos + (N_DEV - 1)) % N_DEV) * 2 + 1)

    out, _ = pl.pallas_call(
        body,
        out_shape=[
            jax.ShapeDtypeStruct((N_DEV * m_per, n_per), jnp.float32),
            jax.ShapeDtypeStruct((N_DEV * m_per, k), x.dtype),
        ],
        in_specs=[
            pl.BlockSpec(memory_space=pl.ANY),
            pl.BlockSpec(memory_space=pl.ANY),
        ],
        out_specs=[
            pl.BlockSpec(memory_space=pl.ANY),
            pl.BlockSpec(memory_space=pl.ANY),
        ],
        scratch_shapes=[
            pltpu.SemaphoreType.DMA,
            pltpu.SemaphoreType.DMA((N_DEV - 1,)),
            pltpu.SemaphoreType.DMA((N_DEV - 1,)),
            pltpu.SemaphoreType.DMA((N_DEV - 1,)),
            pltpu.SemaphoreType.DMA((N_DEV - 1,)),
        ],
        compiler_params=pltpu.CompilerParams(
            collective_id=0,
            vmem_limit_bytes=60 * 1024 * 1024,
        ),
    )(x, w_mat)
    return out


# device time: 334379 ns/iter; 2.5515x vs baseline; 2.5515x over previous
import jax
import jax.numpy as jnp
from jax import lax
from jax.experimental import pallas as pl
from jax.experimental.pallas import tpu as pltpu

N_DEV = 4

_GELU_C = 0.7978845608028654


def _gelu(y):
    return 0.5 * y * (1.0 + jnp.tanh(_GELU_C * (y + 0.044715 * y * y * y)))


def kernel(x, w_mat):
    m_per, k = x.shape
    _, n_per = w_mat.shape
    half = m_per // 2
    blk_n = 512
    n_tiles = n_per // blk_n

    def body(x_ref, w_ref, out_ref, cw_buf, ccw_buf, outv,
             cp_sems, st_sems, cw_send, cw_recv, ccw_send, ccw_recv):
        my_pos = lax.axis_index("i")
        left = (my_pos - 1) % N_DEV
        right = (my_pos + 1) % N_DEV

        barrier_sem = pltpu.get_barrier_semaphore()
        for nbr in [left, right]:
            pl.semaphore_signal(
                barrier_sem, inc=1,
                device_id=(nbr,), device_id_type=pl.DeviceIdType.MESH,
            )
        pl.semaphore_wait(barrier_sem, 2)

        cp_top = pltpu.make_async_copy(
            x_ref.at[pl.ds(0, half), :], cw_buf.at[0], cp_sems.at[0]
        )
        cp_bot = pltpu.make_async_copy(
            x_ref.at[pl.ds(half, half), :], ccw_buf.at[0], cp_sems.at[1]
        )
        cp_top.start()
        cp_bot.start()
        cp_top.wait()
        cp_bot.wait()

        def compute_from(src_ref, slot):
            def inner(w_blk, o_blk):
                y = jnp.dot(
                    src_ref[...], w_blk[...], preferred_element_type=jnp.float32
                )
                o_blk[...] = _gelu(y)

            pipe = pltpu.emit_pipeline(
                inner,
                grid=(n_tiles,),
                in_specs=[pl.BlockSpec((k, blk_n), lambda j: (0, j))],
                out_specs=[pl.BlockSpec((half, blk_n), lambda j: (0, j))],
            )
            pipe(w_ref, outv.at[slot])

        quarter = half // 2

        def compute_q(src_ref, slot, qoff):
            def inner(w_blk, o_blk):
                y = jnp.dot(
                    src_ref[...], w_blk[...], preferred_element_type=jnp.float32
                )
                o_blk[...] = _gelu(y)

            pipe = pltpu.emit_pipeline(
                inner,
                grid=(n_tiles,),
                in_specs=[pl.BlockSpec((k, blk_n), lambda j: (0, j))],
                out_specs=[pl.BlockSpec((quarter, blk_n), lambda j: (0, j))],
            )
            pipe(w_ref, outv.at[slot, pl.ds(qoff, quarter), :])

        def store_q(slot, qoff, out_row, sem_i):
            st = pltpu.make_async_copy(
                outv.at[slot, pl.ds(qoff, quarter), :],
                out_ref.at[pl.ds(out_row, quarter), :],
                st_sems.at[sem_i],
            )
            st.start()
            return st

        def store_rows(slot, row_blk, sem_i):
            st = pltpu.make_async_copy(
                outv.at[slot],
                out_ref.at[pl.ds(row_blk * half, half), :],
                st_sems.at[sem_i],
            )
            st.start()
            return st

        prev_st = None
        for h in range(N_DEV - 2):
            s, r = h % 2, (h + 1) % 2
            rdma_cw = pltpu.make_async_remote_copy(
                src_ref=cw_buf.at[s],
                dst_ref=cw_buf.at[r],
                send_sem=cw_send.at[h],
                recv_sem=cw_recv.at[h],
                device_id=(right,),
                device_id_type=pl.DeviceIdType.MESH,
            )
            rdma_ccw = pltpu.make_async_remote_copy(
                src_ref=ccw_buf.at[s],
                dst_ref=ccw_buf.at[r],
                send_sem=ccw_send.at[h],
                recv_sem=ccw_recv.at[h],
                device_id=(left,),
                device_id_type=pl.DeviceIdType.MESH,
            )
            rdma_cw.start()
            rdma_ccw.start()
            o_cw = (my_pos - h) % N_DEV
            o_ccw = (my_pos + h) % N_DEV
            if prev_st is not None:
                prev_st[0].wait()
                prev_st[1].wait()
            compute_from(cw_buf.at[s], 0)
            compute_from(ccw_buf.at[s], 1)
            rdma_cw.wait()
            rdma_ccw.wait()
            prev_st = (
                store_rows(0, o_cw * 2, 0),
                store_rows(1, o_ccw * 2 + 1, 1),
            )

        h = N_DEV - 2
        s, r = h % 2, (h + 1) % 2
        subs = []
        for q in range(2):
            rd_cw = pltpu.make_async_remote_copy(
                src_ref=cw_buf.at[s, pl.ds(q * quarter, quarter), :],
                dst_ref=cw_buf.at[r, pl.ds(q * quarter, quarter), :],
                send_sem=cw_send.at[h + q],
                recv_sem=cw_recv.at[h + q],
                device_id=(right,),
                device_id_type=pl.DeviceIdType.MESH,
            )
            rd_ccw = pltpu.make_async_remote_copy(
                src_ref=ccw_buf.at[s, pl.ds(q * quarter, quarter), :],
                dst_ref=ccw_buf.at[r, pl.ds(q * quarter, quarter), :],
                send_sem=ccw_send.at[h + q],
                recv_sem=ccw_recv.at[h + q],
                device_id=(left,),
                device_id_type=pl.DeviceIdType.MESH,
            )
            rd_cw.start()
            rd_ccw.start()
            subs.append((rd_cw, rd_ccw))

        o_cw = (my_pos - h) % N_DEV
        o_ccw = (my_pos + h) % N_DEV
        prev_st[0].wait()
        prev_st[1].wait()
        compute_from(cw_buf.at[s], 0)
        compute_from(ccw_buf.at[s], 1)
        st_a = store_rows(0, o_cw * 2, 0)
        st_b = store_rows(1, o_ccw * 2 + 1, 1)

        o3_cw = (my_pos - (N_DEV - 1)) % N_DEV
        o3_ccw = (my_pos + (N_DEV - 1)) % N_DEV
        q_sts = []
        for q in range(2):
            subs[q][0].wait()
            subs[q][1].wait()
            if q == 0:
                st_a.wait()
                st_b.wait()
            compute_q(cw_buf.at[r, pl.ds(q * quarter, quarter), :], 0, q * quarter)
            compute_q(ccw_buf.at[r, pl.ds(q * quarter, quarter), :], 1, q * quarter)
            q_sts.append(store_q(0, q * quarter, o3_cw * m_per + q * quarter, 2 + 2 * q))
            q_sts.append(store_q(1, q * quarter, o3_ccw * m_per + half + q * quarter, 3 + 2 * q))
        for st in q_sts:
            st.wait()

    return pl.pallas_call(
        body,
        out_shape=jax.ShapeDtypeStruct((N_DEV * m_per, n_per), jnp.float32),
        in_specs=[
            pl.BlockSpec(memory_space=pl.ANY),
            pl.BlockSpec(memory_space=pl.ANY),
        ],
        out_specs=pl.BlockSpec(memory_space=pl.ANY),
        scratch_shapes=[
            pltpu.VMEM((2, half, k), jnp.float32),
            pltpu.VMEM((2, half, k), jnp.float32),
            pltpu.VMEM((2, half, n_per), jnp.float32),
            pltpu.SemaphoreType.DMA((2,)),
            pltpu.SemaphoreType.DMA((6,)),
            pltpu.SemaphoreType.DMA((N_DEV,)),
            pltpu.SemaphoreType.DMA((N_DEV,)),
            pltpu.SemaphoreType.DMA((N_DEV,)),
            pltpu.SemaphoreType.DMA((N_DEV,)),
        ],
        compiler_params=pltpu.CompilerParams(
            collective_id=0,
            vmem_limit_bytes=62 * 1024 * 1024,
        ),
    )(x, w_mat)
